# Initial kernel scaffold; baseline (speedup 1.0000x reference)
import jax
import jax.numpy as jnp
from jax.experimental import pallas as pl


def kernel(hidden_states, gate_proj_w, gate_weights, up_weights, down_weights):
    raise NotImplementedError("write your pallas kernel here")


# speedup vs baseline: 1.3581x; 1.3581x over previous
import functools

import jax
import jax.numpy as jnp
from jax.experimental import pallas as pl
from jax.experimental.pallas import tpu as pltpu

T = 2048
H = 1024
I = 1024
E = 8
TOP_K = 2

IB = 512
NI = I // IB
TC_CHUNK = 256


def _moe_body(hidden_ref, gatew_ref, wg_ref, wu_ref, wd_ref, out_ref,
              xbf_ref, comb_ref):
    e = pl.program_id(0)
    i = pl.program_id(1)

    @pl.when(jnp.logical_and(e == 0, i == 0))
    def _prologue():
        x = hidden_ref[...]
        xbf_ref[...] = x.astype(jnp.bfloat16)
        logits = jax.lax.dot_general(
            x, gatew_ref[...], (((1,), (1,)), ((), ())),
            preferred_element_type=jnp.float32)
        m = jnp.max(logits, axis=1, keepdims=True)
        ex = jnp.exp(logits - m)
        p = ex / jnp.sum(ex, axis=1, keepdims=True)
        lane = jax.lax.broadcasted_iota(jnp.int32, (T, E), 1)
        p1 = jnp.max(p, axis=1, keepdims=True)
        i1 = jnp.min(jnp.where(p == p1, lane, E), axis=1, keepdims=True)
        pm = jnp.where(lane == i1, -jnp.inf, p)
        p2 = jnp.max(pm, axis=1, keepdims=True)
        i2 = jnp.min(jnp.where(pm == p2, lane, E), axis=1, keepdims=True)
        denom = p1 + p2
        comb_ref[...] = jnp.where(lane == i1, p1 / denom, 0.0) + \
            jnp.where(lane == i2, p2 / denom, 0.0)
        out_ref[...] = jnp.zeros((T, H), jnp.float32)

    wg = wg_ref[0].astype(jnp.bfloat16)
    wu = wu_ref[0].astype(jnp.bfloat16)
    wd = wd_ref[0].astype(jnp.bfloat16)
    lane = jax.lax.broadcasted_iota(jnp.int32, (TC_CHUNK, E), 1)

    def chunk(tc, _):
        rows = pl.ds(tc * TC_CHUNK, TC_CHUNK)
        x = xbf_ref[rows, :]
        g = jnp.dot(x, wg, preferred_element_type=jnp.float32)
        u = jnp.dot(x, wu, preferred_element_type=jnp.float32)
        act = (g / (1.0 + jnp.exp(-g)) * u).astype(jnp.bfloat16)
        y = jnp.dot(act, wd, preferred_element_type=jnp.float32)
        ce = jnp.sum(jnp.where(lane == e, comb_ref[rows, :], 0.0),
                     axis=1, keepdims=True)
        out_ref[rows, :] += ce * y
        return 0

    jax.lax.fori_loop(0, T // TC_CHUNK, chunk, 0)


@jax.jit
def kernel(hidden_states, gate_proj_w, gate_weights, up_weights, down_weights):
    return pl.pallas_call(
        _moe_body,
        grid=(E, NI),
        in_specs=[
            pl.BlockSpec((T, H), lambda e, i: (0, 0)),
            pl.BlockSpec((E, H), lambda e, i: (0, 0)),
            pl.BlockSpec((1, H, IB), lambda e, i: (e, 0, i)),
            pl.BlockSpec((1, H, IB), lambda e, i: (e, 0, i)),
            pl.BlockSpec((1, IB, H), lambda e, i: (e, i, 0)),
        ],
        out_specs=pl.BlockSpec((T, H), lambda e, i: (0, 0)),
        out_shape=jax.ShapeDtypeStruct((T, H), jnp.float32),
        scratch_shapes=[
            pltpu.VMEM((T, H), jnp.bfloat16),
            pltpu.VMEM((T, E), jnp.float32),
        ],
        compiler_params=pltpu.CompilerParams(
            dimension_semantics=("arbitrary", "arbitrary"),
        ),
    )(hidden_states, gate_proj_w, gate_weights, up_weights, down_weights)


# speedup vs baseline: 1.4013x; 1.0318x over previous
import functools

import jax
import jax.numpy as jnp
from jax.experimental import pallas as pl
from jax.experimental.pallas import tpu as pltpu

T = 2048
H = 1024
I = 1024
E = 8
TOP_K = 2

IB = 512
NI = I // IB
TC_CHUNK = 256


def _moe_body(hidden_ref, gatew_ref, wg_ref, wu_ref, wd_ref, out_ref,
              comb_ref):
    e = pl.program_id(0)
    i = pl.program_id(1)

    @pl.when(jnp.logical_and(e == 0, i == 0))
    def _prologue():
        x = hidden_ref[...]
        logits = jax.lax.dot_general(
            x, gatew_ref[...], (((1,), (1,)), ((), ())),
            preferred_element_type=jnp.float32)
        m = jnp.max(logits, axis=1, keepdims=True)
        ex = jnp.exp(logits - m)
        p = ex / jnp.sum(ex, axis=1, keepdims=True)
        lane = jax.lax.broadcasted_iota(jnp.int32, (T, E), 1)
        p1 = jnp.max(p, axis=1, keepdims=True)
        i1 = jnp.min(jnp.where(p == p1, lane, E), axis=1, keepdims=True)
        pm = jnp.where(lane == i1, -jnp.inf, p)
        p2 = jnp.max(pm, axis=1, keepdims=True)
        i2 = jnp.min(jnp.where(pm == p2, lane, E), axis=1, keepdims=True)
        denom = p1 + p2
        comb_ref[...] = jnp.where(lane == i1, p1 / denom, 0.0) + \
            jnp.where(lane == i2, p2 / denom, 0.0)
        out_ref[...] = jnp.zeros((T, H), jnp.float32)

    wg = wg_ref[0]
    wu = wu_ref[0]
    wd = wd_ref[0]
    lane = jax.lax.broadcasted_iota(jnp.int32, (TC_CHUNK, E), 1)

    def chunk(tc, _):
        rows = pl.ds(tc * TC_CHUNK, TC_CHUNK)
        x = hidden_ref[rows, :]
        g = jnp.dot(x, wg, preferred_element_type=jnp.float32)
        u = jnp.dot(x, wu, preferred_element_type=jnp.float32)
        act = g / (1.0 + jnp.exp(-g)) * u
        y = jnp.dot(act, wd, preferred_element_type=jnp.float32)
        ce = jnp.sum(jnp.where(lane == e, comb_ref[rows, :], 0.0),
                     axis=1, keepdims=True)
        out_ref[rows, :] += ce * y
        return 0

    jax.lax.fori_loop(0, T // TC_CHUNK, chunk, 0)


@jax.jit
def kernel(hidden_states, gate_proj_w, gate_weights, up_weights, down_weights):
    return pl.pallas_call(
        _moe_body,
        grid=(E, NI),
        in_specs=[
            pl.BlockSpec((T, H), lambda e, i: (0, 0)),
            pl.BlockSpec((E, H), lambda e, i: (0, 0)),
            pl.BlockSpec((1, H, IB), lambda e, i: (e, 0, i)),
            pl.BlockSpec((1, H, IB), lambda e, i: (e, 0, i)),
            pl.BlockSpec((1, IB, H), lambda e, i: (e, i, 0)),
        ],
        out_specs=pl.BlockSpec((T, H), lambda e, i: (0, 0)),
        out_shape=jax.ShapeDtypeStruct((T, H), jnp.float32),
        scratch_shapes=[
            pltpu.VMEM((T, E), jnp.float32),
        ],
        compiler_params=pltpu.CompilerParams(
            dimension_semantics=("arbitrary", "arbitrary"),
        ),
    )(hidden_states, gate_proj_w, gate_weights, up_weights, down_weights)
